# Initial kernel scaffold; baseline (speedup 1.0000x reference)
import jax
import jax.numpy as jnp
from jax.experimental import pallas as pl


def kernel(features, edges, Ws0, bs0, Wn0, bn0, Ws1, bs1, Wn1, bn1, Ws2, bs2, Wn2, bn2, Ws3, bs3, Wn3, bn3):
    raise NotImplementedError("write your pallas kernel here")


# speedup vs baseline: 3.9631x; 3.9631x over previous
import functools

import jax
import jax.numpy as jnp
from jax import lax
from jax.experimental import pallas as pl
from jax.experimental.pallas import tpu as pltpu
from jax.experimental.pallas import tpu_sc as plsc

_N = 10000
_D = 128
_E = 320000

_NC = 2
_NS = 16
_NW = _NC * _NS
_CK = 128
_CHUNKS = -(-_E // (_NW * _CK))
_EPAD = _NW * _CHUNKS * _CK
_NPAD = _N + 112
_RPT = _NPAD // _NS


def _sc_body(x_hbm, src_hbm, dst_hbm, zeros_hbm, out_hbm,
             src_v, dst_v, rows_v, agg_sh, sem):
    c = lax.axis_index("c")
    s = lax.axis_index("s")
    wid = c * _NS + s
    pltpu.sync_copy(zeros_hbm.at[pl.ds(s * _RPT, _RPT)],
                    agg_sh.at[pl.ds(s * _RPT, _RPT)])
    pltpu.sync_copy(src_hbm.at[wid], src_v)
    pltpu.sync_copy(dst_hbm.at[wid], dst_v)
    plsc.subcore_barrier()

    def chunk(ci, carry):
        pltpu.async_copy(x_hbm.at[src_v.at[ci]], rows_v, sem).wait()
        pltpu.sync_copy(rows_v, agg_sh.at[dst_v.at[ci]], add=True)
        return carry

    lax.fori_loop(0, _CHUNKS, chunk, 0)
    plsc.subcore_barrier()
    pltpu.sync_copy(agg_sh.at[pl.ds(s * _RPT, _RPT)],
                    out_hbm.at[c, pl.ds(s * _RPT, _RPT)])


_sc_seg = pl.kernel(
    _sc_body,
    out_type=jax.ShapeDtypeStruct((_NC, _NPAD, _D), jnp.float32),
    mesh=plsc.VectorSubcoreMesh(core_axis_name="c", subcore_axis_name="s"),
    scratch_types=[
        pltpu.VMEM((_CHUNKS, _CK), jnp.int32),
        pltpu.VMEM((_CHUNKS, _CK), jnp.int32),
        pltpu.VMEM((_CK, _D), jnp.float32),
        pltpu.VMEM_SHARED((_NPAD, _D), jnp.float32),
        pltpu.SemaphoreType.DMA,
    ],
)


_BLK = 1000


def _dense_nores(x_ref, agg_ref, ws_ref, wn_ref, b_ref, o_ref, *, relu):
    acc = jnp.dot(x_ref[...], ws_ref[...], preferred_element_type=jnp.float32)
    agg = agg_ref[0] + agg_ref[1]
    acc = acc + jnp.dot(agg, wn_ref[...], preferred_element_type=jnp.float32)
    acc = acc + b_ref[...]
    if relu:
        acc = jnp.maximum(acc, 0.0)
    o_ref[...] = acc


def _dense_res(x_ref, agg_ref, ws_ref, wn_ref, b_ref, res_ref, o_ref, *, relu):
    acc = jnp.dot(x_ref[...], ws_ref[...], preferred_element_type=jnp.float32)
    agg = agg_ref[0] + agg_ref[1]
    acc = acc + jnp.dot(agg, wn_ref[...], preferred_element_type=jnp.float32)
    acc = acc + b_ref[...] + res_ref[...]
    if relu:
        acc = jnp.maximum(acc, 0.0)
    o_ref[...] = acc


def _dense_call(x, aggs, Ws, Wn, b, res=None, relu=True):
    in_specs = [
        pl.BlockSpec((_BLK, _D), lambda i: (i, 0)),
        pl.BlockSpec((_NC, _BLK, _D), lambda i: (0, i, 0)),
        pl.BlockSpec((_D, _D), lambda i: (0, 0)),
        pl.BlockSpec((_D, _D), lambda i: (0, 0)),
        pl.BlockSpec((1, _D), lambda i: (0, 0)),
    ]
    args = [x, aggs, Ws, Wn, b.reshape(1, _D)]
    if res is not None:
        in_specs.append(pl.BlockSpec((_BLK, _D), lambda i: (i, 0)))
        args.append(res)
        body = functools.partial(_dense_res, relu=relu)
    else:
        body = functools.partial(_dense_nores, relu=relu)
    return pl.pallas_call(
        body,
        grid=(_N // _BLK,),
        in_specs=in_specs,
        out_specs=pl.BlockSpec((_BLK, _D), lambda i: (i, 0)),
        out_shape=jax.ShapeDtypeStruct((_N, _D), jnp.float32),
    )(*args)


def kernel(features, edges,
           Ws0, bs0, Wn0, bn0,
           Ws1, bs1, Wn1, bn1,
           Ws2, bs2, Wn2, bn2,
           Ws3, bs3, Wn3, bn3):
    pad = _EPAD - _E
    srcp = jnp.concatenate(
        [edges[0], jnp.zeros((pad,), jnp.int32)]).reshape(_NW, _CHUNKS, _CK)
    dstp = jnp.concatenate(
        [edges[1], jnp.full((pad,), _N, jnp.int32)]).reshape(_NW, _CHUNKS, _CK)
    zeros = jnp.zeros((_NPAD, _D), jnp.float32)

    x = features
    layers = ((Ws0, bs0, Wn0, bn0), (Ws1, bs1, Wn1, bn1),
              (Ws2, bs2, Wn2, bn2), (Ws3, bs3, Wn3, bn3))
    for i, (Ws, bs, Wn, bn) in enumerate(layers):
        aggs = _sc_seg(x, srcp, dstp, zeros)
        x = _dense_call(x, aggs, Ws, Wn, bs + bn,
                        res=features if i == 2 else None,
                        relu=(i != 3))
    return x


# speedup vs baseline: 5.7480x; 1.4504x over previous
import functools

import jax
import jax.numpy as jnp
from jax import lax
from jax.experimental import pallas as pl
from jax.experimental.pallas import tpu as pltpu
from jax.experimental.pallas import tpu_sc as plsc

_N = 10000
_D = 128
_E = 320000

_NC = 2
_NS = 16
_NW = _NC * _NS
_CK = 96
_CHUNKS = -(-_E // (_NW * _CK))
_EPW = _CHUNKS * _CK
_EPAD = _NW * _CHUNKS * _CK
_NPAD = _N + 112
_RPT = _NPAD // _NS


_NBUF = 2


def _sc_body(x_hbm, src_hbm, dst_hbm, zeros_hbm, out_hbm,
             src_v, dst_v, rows_v, agg_sh, gsem, ssem):
    c = lax.axis_index("c")
    s = lax.axis_index("s")
    wid = c * _NS + s
    pltpu.sync_copy(zeros_hbm.at[pl.ds(s * _RPT, _RPT)],
                    agg_sh.at[pl.ds(s * _RPT, _RPT)])
    pltpu.sync_copy(src_hbm.at[pl.ds(wid * _EPW, _EPW)], src_v)
    pltpu.sync_copy(dst_hbm.at[wid], dst_v)
    plsc.subcore_barrier()

    def g_start(ci, b):
        pltpu.async_copy(x_hbm.at[src_v.at[pl.ds(ci * _CK, _CK)]],
                         rows_v.at[b], gsem)

    def g_wait(ci, b):
        pltpu.make_async_copy(x_hbm.at[src_v.at[pl.ds(ci * _CK, _CK)]],
                              rows_v.at[b], gsem).wait()

    def s_start(ci, b):
        pltpu.async_copy(rows_v.at[b], agg_sh.at[dst_v.at[ci]], ssem, add=True)

    def s_wait(ci, b):
        pltpu.make_async_copy(rows_v.at[b], agg_sh.at[dst_v.at[ci]],
                              ssem).wait()

    g_start(0, 0)
    g_start(1, 1)

    def chunk(ci, carry):
        b = lax.rem(ci, _NBUF)
        g_wait(ci, b)
        s_start(ci, b)
        s_wait(ci, b)

        @pl.when(ci + 2 < _CHUNKS)
        def _():
            g_start(ci + 2, b)

        return carry

    lax.fori_loop(0, _CHUNKS, chunk, 0)
    plsc.subcore_barrier()
    pltpu.sync_copy(agg_sh.at[pl.ds(s * _RPT, _RPT)],
                    out_hbm.at[c, pl.ds(s * _RPT, _RPT)])


_sc_seg = pl.kernel(
    _sc_body,
    out_type=jax.ShapeDtypeStruct((_NC, _NPAD, _D), jnp.float32),
    mesh=plsc.VectorSubcoreMesh(core_axis_name="c", subcore_axis_name="s"),
    scratch_types=[
        pltpu.VMEM((_EPW,), jnp.int32),
        pltpu.VMEM((_CHUNKS, _CK), jnp.int32),
        pltpu.VMEM((_NBUF, _CK, _D), jnp.float32),
        pltpu.VMEM_SHARED((_NPAD, _D), jnp.float32),
        pltpu.SemaphoreType.DMA,
        pltpu.SemaphoreType.DMA,
    ],
)


_BLK = 1000


def _dense_nores(x_ref, agg_ref, ws_ref, wn_ref, b_ref, o_ref, *, relu):
    acc = jnp.dot(x_ref[...], ws_ref[...], preferred_element_type=jnp.float32)
    agg = agg_ref[0] + agg_ref[1]
    acc = acc + jnp.dot(agg, wn_ref[...], preferred_element_type=jnp.float32)
    acc = acc + b_ref[...]
    if relu:
        acc = jnp.maximum(acc, 0.0)
    o_ref[...] = acc


def _dense_res(x_ref, agg_ref, ws_ref, wn_ref, b_ref, res_ref, o_ref, *, relu):
    acc = jnp.dot(x_ref[...], ws_ref[...], preferred_element_type=jnp.float32)
    agg = agg_ref[0] + agg_ref[1]
    acc = acc + jnp.dot(agg, wn_ref[...], preferred_element_type=jnp.float32)
    acc = acc + b_ref[...] + res_ref[...]
    if relu:
        acc = jnp.maximum(acc, 0.0)
    o_ref[...] = acc


def _dense_call(x, aggs, Ws, Wn, b, res=None, relu=True):
    in_specs = [
        pl.BlockSpec((_BLK, _D), lambda i: (i, 0)),
        pl.BlockSpec((_NC, _BLK, _D), lambda i: (0, i, 0)),
        pl.BlockSpec((_D, _D), lambda i: (0, 0)),
        pl.BlockSpec((_D, _D), lambda i: (0, 0)),
        pl.BlockSpec((1, _D), lambda i: (0, 0)),
    ]
    args = [x, aggs, Ws, Wn, b.reshape(1, _D)]
    if res is not None:
        in_specs.append(pl.BlockSpec((_BLK, _D), lambda i: (i, 0)))
        args.append(res)
        body = functools.partial(_dense_res, relu=relu)
    else:
        body = functools.partial(_dense_nores, relu=relu)
    return pl.pallas_call(
        body,
        grid=(_N // _BLK,),
        in_specs=in_specs,
        out_specs=pl.BlockSpec((_BLK, _D), lambda i: (i, 0)),
        out_shape=jax.ShapeDtypeStruct((_N, _D), jnp.float32),
    )(*args)


def kernel(features, edges,
           Ws0, bs0, Wn0, bn0,
           Ws1, bs1, Wn1, bn1,
           Ws2, bs2, Wn2, bn2,
           Ws3, bs3, Wn3, bn3):
    pad = _EPAD - _E
    srcp = jnp.concatenate([edges[0], jnp.zeros((pad,), jnp.int32)])
    dstp = jnp.concatenate(
        [edges[1], jnp.full((pad,), _N, jnp.int32)]).reshape(_NW, _CHUNKS, _CK)
    zeros = jnp.zeros((_NPAD, _D), jnp.float32)

    x = features
    layers = ((Ws0, bs0, Wn0, bn0), (Ws1, bs1, Wn1, bn1),
              (Ws2, bs2, Wn2, bn2), (Ws3, bs3, Wn3, bn3))
    for i, (Ws, bs, Wn, bn) in enumerate(layers):
        aggs = _sc_seg(x, srcp, dstp, zeros)
        x = _dense_call(x, aggs, Ws, Wn, bs + bn,
                        res=features if i == 2 else None,
                        relu=(i != 3))
    return x
